# baseline (speedup 1.0000x reference)
import functools

import jax
import jax.numpy as jnp
from jax import lax
from jax.experimental import pallas as pl
from jax.experimental.pallas import tpu as pltpu
from jax.experimental.pallas import tpu_sc as plsc

_WD = 0.01
_NC, _NS, _L = 2, 16, 16
_NW = _NC * _NS
_B = 16384
_D = 64
_BPW = _B // _NW
_CH = 128
_NCH = _BPW // _CH


def _sc_body(u_hbm, i_hbm, j_hbm, w_hbm, h_hbm, x_hbm, reg_hbm,
             idx_u, idx_i, idx_j, rows_u, rows_i, rows_j, x_v, reg_v, sem):
    wid = lax.axis_index("s") * _NC + lax.axis_index("c")
    base = wid * _BPW

    pltpu.sync_copy(u_hbm.at[pl.ds(base, _BPW)], idx_u)
    pltpu.sync_copy(i_hbm.at[pl.ds(base, _BPW)], idx_i)
    pltpu.sync_copy(j_hbm.at[pl.ds(base, _BPW)], idx_j)

    copies = []
    for c in range(_NCH):
        sl = pl.ds(c * _CH, _CH)
        copies.append(pltpu.async_copy(w_hbm.at[idx_u.at[sl]], rows_u.at[sl], sem))
        copies.append(pltpu.async_copy(h_hbm.at[idx_i.at[sl]], rows_i.at[sl], sem))
        copies.append(pltpu.async_copy(h_hbm.at[idx_j.at[sl]], rows_j.at[sl], sem))
    for cp in copies:
        cp.wait()

    def row_body(r, reg_acc):
        pd = jnp.zeros((_L,), jnp.float32)
        for c in range(_D // _L):
            sl = pl.ds(c * _L, _L)
            ue = rows_u[r, sl]
            ie = rows_i[r, sl]
            je = rows_j[r, sl]
            pd = pd + ue * (ie - je)
            reg_acc = reg_acc + ue * ue + ie * ie + je * je
        x_v[r] = pd
        return reg_acc

    reg_acc = lax.fori_loop(0, _BPW, row_body, jnp.zeros((_L,), jnp.float32))
    reg_v[...] = reg_acc

    pltpu.sync_copy(x_v, x_hbm.at[pl.ds(base, _BPW)])
    pltpu.sync_copy(reg_v, reg_hbm.at[wid])


def _sc_call(u, i, j, W, H):
    mesh = plsc.VectorSubcoreMesh(core_axis_name="c", subcore_axis_name="s")
    return pl.kernel(
        _sc_body,
        out_type=(
            jax.ShapeDtypeStruct((_B, _L), jnp.float32),
            jax.ShapeDtypeStruct((_NW, _L), jnp.float32),
        ),
        mesh=mesh,
        scratch_types=[
            pltpu.VMEM((_BPW,), jnp.int32),
            pltpu.VMEM((_BPW,), jnp.int32),
            pltpu.VMEM((_BPW,), jnp.int32),
            pltpu.VMEM((_BPW, _D), jnp.float32),
            pltpu.VMEM((_BPW, _D), jnp.float32),
            pltpu.VMEM((_BPW, _D), jnp.float32),
            pltpu.VMEM((_BPW, _L), jnp.float32),
            pltpu.VMEM((_L,), jnp.float32),
            pltpu.SemaphoreType.DMA,
        ],
        compiler_params=pltpu.CompilerParams(use_tc_tiling_on_sc=False),
    )(u, i, j, W, H)


def _tc_body(x2_ref, reg_ref, o_ref):
    v = x2_ref[...]
    rt = lax.broadcasted_iota(jnp.int32, (128, 8), 0) // _L
    ct = lax.broadcasted_iota(jnp.int32, (128, 8), 1)
    m8 = (rt == ct).astype(jnp.float32)
    x = jnp.dot(v, m8, preferred_element_type=jnp.float32)
    ls = jnp.minimum(x, 0.0) - jnp.log1p(jnp.exp(-jnp.abs(x)))
    val = -jnp.sum(ls) + _WD * jnp.sum(reg_ref[...])
    o_ref[...] = jnp.broadcast_to(val, (1, 1))


def _tc_call(x2, reg):
    return pl.pallas_call(
        _tc_body,
        out_shape=jax.ShapeDtypeStruct((1, 1), jnp.float32),
    )(x2, reg)


def kernel(u, i, j, W, H):
    u = u.astype(jnp.int32)
    i = i.astype(jnp.int32)
    j = j.astype(jnp.int32)
    x2, reg = _sc_call(u, i, j, W, H)
    out = _tc_call(x2.reshape(_B * _L // 128, 128), reg)
    return out[0, 0]


# speedup vs baseline: 1.3407x; 1.3407x over previous
import functools

import jax
import jax.numpy as jnp
from jax import lax
from jax.experimental import pallas as pl
from jax.experimental.pallas import tpu as pltpu
from jax.experimental.pallas import tpu_sc as plsc

_WD = 0.01
_NC, _NS, _L = 2, 16, 16
_NW = _NC * _NS
_B = 16384
_D = 64
_BPW = _B // _NW
_HALF = 256


def _sc_body(u_hbm, i_hbm, j_hbm, w_hbm, h_hbm, x_hbm, reg_hbm,
             idx_u, idx_i, idx_j, rows_u, rows_i, rows_j, x_v, reg_v, sem):
    wid = lax.axis_index("s") * _NC + lax.axis_index("c")
    base = wid * _BPW

    pltpu.sync_copy(u_hbm.at[pl.ds(base, _BPW)], idx_u)
    pltpu.sync_copy(i_hbm.at[pl.ds(base, _BPW)], idx_i)
    pltpu.sync_copy(j_hbm.at[pl.ds(base, _BPW)], idx_j)

    reg_acc0 = jnp.zeros((_L,), jnp.float32)

    def compute_half(h, reg_acc):
        def issue_body(t, c):
            vu = idx_u[pl.ds(h * _HALF + t * _L, _L)]
            vi = idx_i[pl.ds(h * _HALF + t * _L, _L)]
            vj = idx_j[pl.ds(h * _HALF + t * _L, _L)]
            for k in range(_L):
                n = t * _L + k
                pltpu.async_copy(w_hbm.at[pl.ds(vu[k], 1)],
                                 rows_u.at[pl.ds(n, 1)], sem)
                pltpu.async_copy(h_hbm.at[pl.ds(vi[k], 1)],
                                 rows_i.at[pl.ds(n, 1)], sem)
                pltpu.async_copy(h_hbm.at[pl.ds(vj[k], 1)],
                                 rows_j.at[pl.ds(n, 1)], sem)
            return c

        lax.fori_loop(0, _HALF // _L, issue_body, 0)
        pltpu.make_async_copy(w_hbm.at[pl.ds(0, _HALF)], rows_u, sem).wait()
        pltpu.make_async_copy(h_hbm.at[pl.ds(0, _HALF)], rows_i, sem).wait()
        pltpu.make_async_copy(h_hbm.at[pl.ds(0, _HALF)], rows_j, sem).wait()

        def row_body(n, acc):
            pd = jnp.zeros((_L,), jnp.float32)
            for c in range(_D // _L):
                sl = pl.ds(c * _L, _L)
                ue = rows_u[n, sl]
                ie = rows_i[n, sl]
                je = rows_j[n, sl]
                pd = pd + ue * (ie - je)
                acc = acc + ue * ue + ie * ie + je * je
            x_v[h * (_HALF // 8) + (n // 8), pl.ds((n % 8) * _L, _L)] = pd
            return acc

        return lax.fori_loop(0, _HALF, row_body, reg_acc)

    for h in range(_BPW // _HALF):
        reg_acc0 = compute_half(h, reg_acc0)

    zeros = jnp.zeros((_L,), jnp.float32)
    for r in range(8):
        for s in range(128 // _L):
            reg_v[r, pl.ds(s * _L, _L)] = zeros
    reg_v[0, pl.ds(0, _L)] = reg_acc0

    pltpu.sync_copy(x_v, x_hbm.at[pl.ds(wid * (_BPW * _L // 128), _BPW * _L // 128)])
    pltpu.sync_copy(reg_v, reg_hbm.at[pl.ds(wid * 8, 8)])


def _sc_call(u, i, j, W, H):
    mesh = plsc.VectorSubcoreMesh(core_axis_name="c", subcore_axis_name="s")
    return pl.kernel(
        _sc_body,
        out_type=(
            jax.ShapeDtypeStruct((_B * _L // 128, 128), jnp.float32),
            jax.ShapeDtypeStruct((_NW * 8, 128), jnp.float32),
        ),
        mesh=mesh,
        scratch_types=[
            pltpu.VMEM((_BPW,), jnp.int32),
            pltpu.VMEM((_BPW,), jnp.int32),
            pltpu.VMEM((_BPW,), jnp.int32),
            pltpu.VMEM((_HALF, _D), jnp.float32),
            pltpu.VMEM((_HALF, _D), jnp.float32),
            pltpu.VMEM((_HALF, _D), jnp.float32),
            pltpu.VMEM((_BPW * _L // 128, 128), jnp.float32),
            pltpu.VMEM((8, 128), jnp.float32),
            pltpu.SemaphoreType.DMA,
        ],
    )(u, i, j, W, H)


def _tc_body(x2_ref, reg_ref, o_ref):
    v = x2_ref[...]
    rt = lax.broadcasted_iota(jnp.int32, (128, 8), 0) // _L
    ct = lax.broadcasted_iota(jnp.int32, (128, 8), 1)
    m8 = (rt == ct).astype(jnp.float32)
    x = jnp.dot(v, m8, preferred_element_type=jnp.float32)
    ls = jnp.minimum(x, 0.0) - jnp.log1p(jnp.exp(-jnp.abs(x)))
    val = -jnp.sum(ls) + _WD * jnp.sum(reg_ref[...])
    o_ref[...] = jnp.broadcast_to(val, (1, 1))


def _tc_call(x2, reg):
    return pl.pallas_call(
        _tc_body,
        out_shape=jax.ShapeDtypeStruct((1, 1), jnp.float32),
    )(x2, reg)


def kernel(u, i, j, W, H):
    u = u.astype(jnp.int32)
    i = i.astype(jnp.int32)
    j = j.astype(jnp.int32)
    x2, reg = _sc_call(u, i, j, W, H)
    out = _tc_call(x2, reg)
    return out[0, 0]
